# Initial kernel scaffold; baseline (speedup 1.0000x reference)
import jax
import jax.numpy as jnp
from jax.experimental import pallas as pl


def kernel(x, edge_index, edge_weight, categories_value, params):
    raise NotImplementedError("write your pallas kernel here")


# speedup vs baseline: 1.8573x; 1.8573x over previous
import functools
import jax
import jax.numpy as jnp
from jax import lax
from jax.experimental import pallas as pl
from jax.experimental.pallas import tpu as pltpu

NN = 50000
EE = 800000
NP = 50176
TILE = 512
GRID = NP // TILE
D_ID, D_H, D_E, D_ALL, HID = 16, 32, 24, 72, 32
LN_EPS = 1e-5


def _elu(v):
    return jnp.where(v > 0, v, jnp.exp(jnp.minimum(v, 0.0)) - 1.0)


def _dinv_from_degp(degp_blk):
    d = jnp.sum(degp_blk, axis=1, keepdims=True)
    return jnp.where(d > 0, lax.rsqrt(jnp.maximum(d, 1e-30)), 0.0)


def _front_body(x_ref, idr_ref, e0_ref, e1_ref, e2_ref, degp_ref,
                wid_ref, bid_ref, wemb_ref, bemb_ref, w0_ref, b0_ref,
                g_ref, b_ref, tagw_ref,
                base0_ref, base1_ref, base2_ref, u3_ref):
    h = _elu(jnp.dot(x_ref[...], w0_ref[...],
                     preferred_element_type=jnp.float32) + b0_ref[...])
    idp = _elu(jnp.dot(idr_ref[...], wid_ref[...],
                       preferred_element_type=jnp.float32) + bid_ref[...])
    wemb = wemb_ref[...]
    ep = (jnp.dot(e0_ref[...], wemb[0:8, :], preferred_element_type=jnp.float32)
          + jnp.dot(e1_ref[...], wemb[8:16, :], preferred_element_type=jnp.float32)
          + jnp.dot(e2_ref[...], wemb[16:24, :], preferred_element_type=jnp.float32)
          + bemb_ref[...])
    ep = _elu(ep)
    s1 = (jnp.sum(idp, axis=1, keepdims=True)
          + jnp.sum(h, axis=1, keepdims=True)
          + jnp.sum(ep, axis=1, keepdims=True))
    s2 = (jnp.sum(idp * idp, axis=1, keepdims=True)
          + jnp.sum(h * h, axis=1, keepdims=True)
          + jnp.sum(ep * ep, axis=1, keepdims=True))
    mu = s1 / D_ALL
    var = s2 / D_ALL - mu * mu
    inv = lax.rsqrt(jnp.maximum(var, 0.0) + LN_EPS)
    g = g_ref[...]
    bb = b_ref[...]
    idn = (idp - mu) * inv * g[:, 0:16] + bb[:, 0:16]
    hn = (h - mu) * inv * g[:, 16:48] + bb[:, 16:48]
    en = (ep - mu) * inv * g[:, 48:72] + bb[:, 48:72]
    tw = tagw_ref[...]

    def base(k):
        return (jnp.dot(idn, tw[k, 0:16, :], preferred_element_type=jnp.float32)
                + jnp.dot(hn, tw[k, 16:48, :], preferred_element_type=jnp.float32)
                + jnp.dot(en, tw[k, 48:72, :], preferred_element_type=jnp.float32))

    base0_ref[...] = base(0)
    base1_ref[...] = base(1)
    base2_ref[...] = base(2)
    dinv = _dinv_from_degp(degp_ref[...])
    u3_ref[...] = dinv * base(3)


def _front(x, idr, e0, e1, e2, degp, p):
    row = lambda i: (i, 0)
    whole2 = lambda shape: pl.BlockSpec(shape, lambda i: (0, 0))
    whole3 = lambda shape: pl.BlockSpec(shape, lambda i: (0, 0, 0))
    out32 = jax.ShapeDtypeStruct((NP, HID), jnp.float32)
    return pl.pallas_call(
        _front_body,
        grid=(GRID,),
        in_specs=[
            pl.BlockSpec((TILE, 16), row),
            pl.BlockSpec((TILE, 16), row),
            pl.BlockSpec((TILE, 8), row),
            pl.BlockSpec((TILE, 8), row),
            pl.BlockSpec((TILE, 8), row),
            pl.BlockSpec((TILE, 2), row),
            whole2((16, 16)), whole2((1, 16)),
            whole2((24, 24)), whole2((1, 24)),
            whole2((16, 32)), whole2((1, 32)),
            whole2((1, 72)), whole2((1, 72)),
            whole3((4, 72, 32)),
        ],
        out_specs=[pl.BlockSpec((TILE, HID), row)] * 4,
        out_shape=[out32] * 4,
    )(x, idr, e0, e1, e2, degp,
      p['W_id'], p['b_id'].reshape(1, -1),
      p['W_emb'], p['b_emb'].reshape(1, -1),
      p['W0'], p['b0'].reshape(1, -1),
      p['ln0_g'].reshape(1, -1), p['ln0_b'].reshape(1, -1),
      p['tag_W'])


def _hop_body(pp_ref, base_ref, degp_ref, u_ref):
    dinv = _dinv_from_degp(degp_ref[...])
    psum = pp_ref[0] + pp_ref[1]
    u_ref[...] = dinv * (base_ref[...] + dinv * psum)


def _hop_combine(pp, base, degp):
    return pl.pallas_call(
        _hop_body,
        grid=(GRID,),
        in_specs=[
            pl.BlockSpec((2, TILE, HID), lambda i: (0, i, 0)),
            pl.BlockSpec((TILE, HID), lambda i: (i, 0)),
            pl.BlockSpec((TILE, 2), lambda i: (i, 0)),
        ],
        out_specs=pl.BlockSpec((TILE, HID), lambda i: (i, 0)),
        out_shape=jax.ShapeDtypeStruct((NP, HID), jnp.float32),
    )(pp, base, degp)


def _epi_body(pp_ref, base_ref, degp_ref, tagb_ref, g_ref, b_ref,
              w1_ref, b1_ref, out_ref):
    dinv = _dinv_from_degp(degp_ref[...])
    y = base_ref[...] + dinv * (pp_ref[0] + pp_ref[1])
    y = jnp.maximum(y + tagb_ref[...], 0.0)
    mu = jnp.mean(y, axis=1, keepdims=True)
    var = jnp.mean(y * y, axis=1, keepdims=True) - mu * mu
    y = (y - mu) * lax.rsqrt(jnp.maximum(var, 0.0) + LN_EPS) * g_ref[...] + b_ref[...]
    o = jnp.dot(y, w1_ref[...], preferred_element_type=jnp.float32) + b1_ref[...]
    m = jnp.max(o, axis=1, keepdims=True)
    z = o - m
    out_ref[...] = z - jnp.log(jnp.sum(jnp.exp(z), axis=1, keepdims=True))


def _epilogue(pp, base0, degp, p):
    whole2 = lambda shape: pl.BlockSpec(shape, lambda i: (0, 0))
    return pl.pallas_call(
        _epi_body,
        grid=(GRID,),
        in_specs=[
            pl.BlockSpec((2, TILE, HID), lambda i: (0, i, 0)),
            pl.BlockSpec((TILE, HID), lambda i: (i, 0)),
            pl.BlockSpec((TILE, 2), lambda i: (i, 0)),
            whole2((1, 32)), whole2((1, 32)), whole2((1, 32)),
            whole2((32, 2)), whole2((1, 2)),
        ],
        out_specs=pl.BlockSpec((TILE, 2), lambda i: (i, 0)),
        out_shape=jax.ShapeDtypeStruct((NP, 2), jnp.float32),
    )(pp, base0, degp,
      p['tag_b'].reshape(1, -1), p['ln1_g'].reshape(1, -1),
      p['ln1_b'].reshape(1, -1), p['W1'], p['b1'].reshape(1, -1))


def kernel(x, edge_index, edge_weight, categories_value, params):
    p = params
    src = edge_index[0]
    dst = edge_index[1]

    idr = jnp.take(p['id_table'], categories_value[:, 0], axis=0)
    e0 = jnp.take(p['emb_tables'][0], categories_value[:, 1], axis=0)
    e1 = jnp.take(p['emb_tables'][1], categories_value[:, 2], axis=0)
    e2 = jnp.take(p['emb_tables'][2], categories_value[:, 3], axis=0)
    deg = jnp.zeros((NN,), jnp.float32).at[dst].add(edge_weight)
    degp = jnp.stack([deg, jnp.zeros_like(deg)], axis=1)

    pad_rows = lambda a: jnp.pad(a, ((0, NP - NN), (0, 0)))
    x_p = pad_rows(x)
    idr_p, e0_p, e1_p, e2_p = map(pad_rows, (idr, e0, e1, e2))
    degp_p = pad_rows(degp)

    base0, base1, base2, u = _front(x_p, idr_p, e0_p, e1_p, e2_p, degp_p, p)

    def scatter_placeholder(u_cur):
        pfull = jnp.zeros((NP, HID), jnp.float32).at[dst].add(
            edge_weight[:, None] * jnp.take(u_cur, src, axis=0))
        return jnp.stack([pfull, jnp.zeros_like(pfull)], axis=0)

    pp = scatter_placeholder(u)
    u = _hop_combine(pp, base2, degp_p)
    pp = scatter_placeholder(u)
    u = _hop_combine(pp, base1, degp_p)
    pp = scatter_placeholder(u)
    out = _epilogue(pp, base0, degp_p, p)
    return out[:NN]


# speedup vs baseline: 7.0740x; 3.8087x over previous
import functools
import jax
import jax.numpy as jnp
from jax import lax
from jax.experimental import pallas as pl
from jax.experimental.pallas import tpu as pltpu
from jax.experimental.pallas import tpu_sc as plsc

NN = 50000
EE = 800000
NP = 50176
TILE = 512
GRID = NP // TILE
D_ID, D_H, D_E, D_ALL, HID = 16, 32, 24, 72, 32
LN_EPS = 1e-5

NTILES = 32
CHUNK = 128
CPT = 200
EP = NTILES * CPT * CHUNK
RPS = NP // 16
ZROWS = 112


def _elu(v):
    return jnp.where(v > 0, v, jnp.exp(jnp.minimum(v, 0.0)) - 1.0)


def _dinv_from_degp(degp_blk):
    d = jnp.sum(degp_blk, axis=1, keepdims=True)
    return jnp.where(d > 0, lax.rsqrt(jnp.maximum(d, 1e-30)), 0.0)


def _front_body(x_ref, idr_ref, e0_ref, e1_ref, e2_ref, degp_ref,
                wid_ref, bid_ref, wemb_ref, bemb_ref, w0_ref, b0_ref,
                g_ref, b_ref, tagw_ref,
                base0_ref, base1_ref, base2_ref, u3_ref):
    h = _elu(jnp.dot(x_ref[...], w0_ref[...],
                     preferred_element_type=jnp.float32) + b0_ref[...])
    idp = _elu(jnp.dot(idr_ref[...], wid_ref[...],
                       preferred_element_type=jnp.float32) + bid_ref[...])
    wemb = wemb_ref[...]
    ep = (jnp.dot(e0_ref[...], wemb[0:8, :], preferred_element_type=jnp.float32)
          + jnp.dot(e1_ref[...], wemb[8:16, :], preferred_element_type=jnp.float32)
          + jnp.dot(e2_ref[...], wemb[16:24, :], preferred_element_type=jnp.float32)
          + bemb_ref[...])
    ep = _elu(ep)
    s1 = (jnp.sum(idp, axis=1, keepdims=True)
          + jnp.sum(h, axis=1, keepdims=True)
          + jnp.sum(ep, axis=1, keepdims=True))
    s2 = (jnp.sum(idp * idp, axis=1, keepdims=True)
          + jnp.sum(h * h, axis=1, keepdims=True)
          + jnp.sum(ep * ep, axis=1, keepdims=True))
    mu = s1 / D_ALL
    var = s2 / D_ALL - mu * mu
    inv = lax.rsqrt(jnp.maximum(var, 0.0) + LN_EPS)
    g = g_ref[...]
    bb = b_ref[...]
    idn = (idp - mu) * inv * g[:, 0:16] + bb[:, 0:16]
    hn = (h - mu) * inv * g[:, 16:48] + bb[:, 16:48]
    en = (ep - mu) * inv * g[:, 48:72] + bb[:, 48:72]
    tw = tagw_ref[...]

    def base(k):
        return (jnp.dot(idn, tw[k, 0:16, :], preferred_element_type=jnp.float32)
                + jnp.dot(hn, tw[k, 16:48, :], preferred_element_type=jnp.float32)
                + jnp.dot(en, tw[k, 48:72, :], preferred_element_type=jnp.float32))

    base0_ref[...] = base(0)
    base1_ref[...] = base(1)
    base2_ref[...] = base(2)
    dinv = _dinv_from_degp(degp_ref[...])
    u3_ref[...] = dinv * base(3)


def _front(x, idr, e0, e1, e2, degp, p):
    row = lambda i: (i, 0)
    whole2 = lambda shape: pl.BlockSpec(shape, lambda i: (0, 0))
    whole3 = lambda shape: pl.BlockSpec(shape, lambda i: (0, 0, 0))
    out32 = jax.ShapeDtypeStruct((NP, HID), jnp.float32)
    return pl.pallas_call(
        _front_body,
        grid=(GRID,),
        in_specs=[
            pl.BlockSpec((TILE, 16), row),
            pl.BlockSpec((TILE, 16), row),
            pl.BlockSpec((TILE, 8), row),
            pl.BlockSpec((TILE, 8), row),
            pl.BlockSpec((TILE, 8), row),
            pl.BlockSpec((TILE, 2), row),
            whole2((16, 16)), whole2((1, 16)),
            whole2((24, 24)), whole2((1, 24)),
            whole2((16, 32)), whole2((1, 32)),
            whole2((1, 72)), whole2((1, 72)),
            whole3((4, 72, 32)),
        ],
        out_specs=[pl.BlockSpec((TILE, HID), row)] * 4,
        out_shape=[out32] * 4,
    )(x, idr, e0, e1, e2, degp,
      p['W_id'], p['b_id'].reshape(1, -1),
      p['W_emb'], p['b_emb'].reshape(1, -1),
      p['W0'], p['b0'].reshape(1, -1),
      p['ln0_g'].reshape(1, -1), p['ln0_b'].reshape(1, -1),
      p['tag_W'])


def _hop_body(pp_ref, base_ref, degp_ref, u_ref):
    dinv = _dinv_from_degp(degp_ref[...])
    psum = pp_ref[0] + pp_ref[1]
    u_ref[...] = dinv * (base_ref[...] + dinv * psum)


def _hop_combine(pp, base, degp):
    return pl.pallas_call(
        _hop_body,
        grid=(GRID,),
        in_specs=[
            pl.BlockSpec((2, TILE, HID), lambda i: (0, i, 0)),
            pl.BlockSpec((TILE, HID), lambda i: (i, 0)),
            pl.BlockSpec((TILE, 2), lambda i: (i, 0)),
        ],
        out_specs=pl.BlockSpec((TILE, HID), lambda i: (i, 0)),
        out_shape=jax.ShapeDtypeStruct((NP, HID), jnp.float32),
    )(pp, base, degp)


def _epi_body(pp_ref, base_ref, degp_ref, tagb_ref, g_ref, b_ref,
              w1_ref, b1_ref, out_ref):
    dinv = _dinv_from_degp(degp_ref[...])
    y = base_ref[...] + dinv * (pp_ref[0] + pp_ref[1])
    y = jnp.maximum(y + tagb_ref[...], 0.0)
    mu = jnp.mean(y, axis=1, keepdims=True)
    var = jnp.mean(y * y, axis=1, keepdims=True) - mu * mu
    y = (y - mu) * lax.rsqrt(jnp.maximum(var, 0.0) + LN_EPS) * g_ref[...] + b_ref[...]
    o = jnp.dot(y, w1_ref[...], preferred_element_type=jnp.float32) + b1_ref[...]
    m = jnp.max(o, axis=1, keepdims=True)
    z = o - m
    out_ref[...] = z - jnp.log(jnp.sum(jnp.exp(z), axis=1, keepdims=True))


def _epilogue(pp, base0, degp, p):
    whole2 = lambda shape: pl.BlockSpec(shape, lambda i: (0, 0))
    return pl.pallas_call(
        _epi_body,
        grid=(GRID,),
        in_specs=[
            pl.BlockSpec((2, TILE, HID), lambda i: (0, i, 0)),
            pl.BlockSpec((TILE, HID), lambda i: (i, 0)),
            pl.BlockSpec((TILE, 2), lambda i: (i, 0)),
            whole2((1, 32)), whole2((1, 32)), whole2((1, 32)),
            whole2((32, 2)), whole2((1, 2)),
        ],
        out_specs=pl.BlockSpec((TILE, 2), lambda i: (i, 0)),
        out_shape=jax.ShapeDtypeStruct((NP, 2), jnp.float32),
    )(pp, base0, degp,
      p['tag_b'].reshape(1, -1), p['ln1_g'].reshape(1, -1),
      p['ln1_b'].reshape(1, -1), p['W1'], p['b1'].reshape(1, -1))


def _hop_sc_body(u_hbm, srcg_hbm, dstg_hbm, wg_hbm, p_hbm,
                 srci, dsti, wi, rows, zbuf, acc, sem):
    core = lax.axis_index("c")
    sub = lax.axis_index("s")
    tile = core * 16 + sub

    zero16 = jnp.zeros((16,), jnp.float32)

    @pl.loop(0, ZROWS)
    def _(r):
        zbuf[r, pl.ds(0, 16)] = zero16
        zbuf[r, pl.ds(16, 16)] = zero16

    @pl.loop(0, RPS // ZROWS)
    def _(t):
        pltpu.sync_copy(zbuf, acc.at[pl.ds(sub * RPS + t * ZROWS, ZROWS)])

    plsc.subcore_barrier()

    @pl.loop(0, CPT // 8)
    def _(jj):
        base = tile * CPT + jj * 8
        pltpu.sync_copy(srcg_hbm.at[pl.ds(base, 8)], srci)
        pltpu.sync_copy(dstg_hbm.at[pl.ds(base, 8)], dsti)
        pltpu.sync_copy(wg_hbm.at[pl.ds(base, 8)], wi)

        @pl.loop(0, 8)
        def _(jr):
            pltpu.async_copy(u_hbm.at[srci.at[jr]], rows, sem).wait()

            @pl.loop(0, CHUNK)
            def _(c):
                s16 = plsc.load_gather(
                    wi, [jnp.full((16,), jr, jnp.int32),
                         jnp.full((16,), c, jnp.int32)])
                rows[c, pl.ds(0, 16)] = rows[c, pl.ds(0, 16)] * s16
                rows[c, pl.ds(16, 16)] = rows[c, pl.ds(16, 16)] * s16

            pltpu.sync_copy(rows, acc.at[dsti.at[jr]], add=True)

    plsc.subcore_barrier()
    pltpu.sync_copy(acc.at[pl.ds(sub * RPS, RPS)],
                    p_hbm.at[core].at[pl.ds(sub * RPS, RPS)])


_hop_sc = pl.kernel(
    _hop_sc_body,
    out_type=jax.ShapeDtypeStruct((2, NP, HID), jnp.float32),
    mesh=plsc.VectorSubcoreMesh(core_axis_name="c", subcore_axis_name="s",
                                num_cores=2, num_subcores=16),
    scratch_types=[
        pltpu.VMEM((8, CHUNK), jnp.int32),
        pltpu.VMEM((8, CHUNK), jnp.int32),
        pltpu.VMEM((8, CHUNK), jnp.float32),
        pltpu.VMEM((CHUNK, HID), jnp.float32),
        pltpu.VMEM((ZROWS, HID), jnp.float32),
        pltpu.VMEM_SHARED((NP, HID), jnp.float32),
        pltpu.SemaphoreType.DMA,
    ],
    compiler_params=pltpu.CompilerParams(use_tc_tiling_on_sc=False,
                                         needs_layout_passes=False),
)


def kernel(x, edge_index, edge_weight, categories_value, params):
    p = params
    src = edge_index[0]
    dst = edge_index[1]

    idr = jnp.take(p['id_table'], categories_value[:, 0], axis=0)
    e0 = jnp.take(p['emb_tables'][0], categories_value[:, 1], axis=0)
    e1 = jnp.take(p['emb_tables'][1], categories_value[:, 2], axis=0)
    e2 = jnp.take(p['emb_tables'][2], categories_value[:, 3], axis=0)
    deg = jnp.zeros((NN,), jnp.float32).at[dst].add(edge_weight)
    degp = jnp.stack([deg, jnp.zeros_like(deg)], axis=1)

    pad_rows = lambda a: jnp.pad(a, ((0, NP - NN), (0, 0)))
    x_p = pad_rows(x)
    idr_p, e0_p, e1_p, e2_p = map(pad_rows, (idr, e0, e1, e2))
    degp_p = pad_rows(degp)

    base0, base1, base2, u = _front(x_p, idr_p, e0_p, e1_p, e2_p, degp_p, p)

    pad_e = lambda a: jnp.pad(a, (0, EP - EE)).reshape(EP // CHUNK, CHUNK)
    srcg = pad_e(src)
    dstg = pad_e(dst)
    wg = pad_e(edge_weight)

    pp = _hop_sc(u, srcg, dstg, wg)
    u = _hop_combine(pp, base2, degp_p)
    pp = _hop_sc(u, srcg, dstg, wg)
    u = _hop_combine(pp, base1, degp_p)
    pp = _hop_sc(u, srcg, dstg, wg)
    out = _epilogue(pp, base0, degp_p, p)
    return out[:NN]


# speedup vs baseline: 7.5515x; 1.0675x over previous
import functools
import jax
import jax.numpy as jnp
from jax import lax
from jax.experimental import pallas as pl
from jax.experimental.pallas import tpu as pltpu
from jax.experimental.pallas import tpu_sc as plsc

NN = 50000
EE = 800000
NP = 50176
TILE = 512
GRID = NP // TILE
D_ID, D_H, D_E, D_ALL, HID = 16, 32, 24, 72, 32
LN_EPS = 1e-5

NTILES = 32
CHUNK = 128
CPT = 200
EP = NTILES * CPT * CHUNK
RPS = NP // 16
ZROWS = 112
BLK = 40


def _elu(v):
    return jnp.where(v > 0, v, jnp.exp(jnp.minimum(v, 0.0)) - 1.0)


def _dinv_from_degp(degp_blk):
    d = jnp.sum(degp_blk, axis=1, keepdims=True)
    return jnp.where(d > 0, lax.rsqrt(jnp.maximum(d, 1e-30)), 0.0)


def _front_body(x_ref, idr_ref, e0_ref, e1_ref, e2_ref, degp_ref,
                wid_ref, bid_ref, wemb_ref, bemb_ref, w0_ref, b0_ref,
                g_ref, b_ref, tagw_ref,
                base0_ref, base1_ref, base2_ref, u3_ref):
    h = _elu(jnp.dot(x_ref[...], w0_ref[...],
                     preferred_element_type=jnp.float32) + b0_ref[...])
    idp = _elu(jnp.dot(idr_ref[...], wid_ref[...],
                       preferred_element_type=jnp.float32) + bid_ref[...])
    wemb = wemb_ref[...]
    ep = (jnp.dot(e0_ref[...], wemb[0:8, :], preferred_element_type=jnp.float32)
          + jnp.dot(e1_ref[...], wemb[8:16, :], preferred_element_type=jnp.float32)
          + jnp.dot(e2_ref[...], wemb[16:24, :], preferred_element_type=jnp.float32)
          + bemb_ref[...])
    ep = _elu(ep)
    s1 = (jnp.sum(idp, axis=1, keepdims=True)
          + jnp.sum(h, axis=1, keepdims=True)
          + jnp.sum(ep, axis=1, keepdims=True))
    s2 = (jnp.sum(idp * idp, axis=1, keepdims=True)
          + jnp.sum(h * h, axis=1, keepdims=True)
          + jnp.sum(ep * ep, axis=1, keepdims=True))
    mu = s1 / D_ALL
    var = s2 / D_ALL - mu * mu
    inv = lax.rsqrt(jnp.maximum(var, 0.0) + LN_EPS)
    g = g_ref[...]
    bb = b_ref[...]
    idn = (idp - mu) * inv * g[:, 0:16] + bb[:, 0:16]
    hn = (h - mu) * inv * g[:, 16:48] + bb[:, 16:48]
    en = (ep - mu) * inv * g[:, 48:72] + bb[:, 48:72]
    tw = tagw_ref[...]

    def base(k):
        return (jnp.dot(idn, tw[k, 0:16, :], preferred_element_type=jnp.float32)
                + jnp.dot(hn, tw[k, 16:48, :], preferred_element_type=jnp.float32)
                + jnp.dot(en, tw[k, 48:72, :], preferred_element_type=jnp.float32))

    base0_ref[...] = base(0)
    base1_ref[...] = base(1)
    base2_ref[...] = base(2)
    dinv = _dinv_from_degp(degp_ref[...])
    u3_ref[...] = dinv * base(3)


def _front(x, idr, e0, e1, e2, degp, p):
    row = lambda i: (i, 0)
    whole2 = lambda shape: pl.BlockSpec(shape, lambda i: (0, 0))
    whole3 = lambda shape: pl.BlockSpec(shape, lambda i: (0, 0, 0))
    out32 = jax.ShapeDtypeStruct((NP, HID), jnp.float32)
    return pl.pallas_call(
        _front_body,
        grid=(GRID,),
        in_specs=[
            pl.BlockSpec((TILE, 16), row),
            pl.BlockSpec((TILE, 16), row),
            pl.BlockSpec((TILE, 8), row),
            pl.BlockSpec((TILE, 8), row),
            pl.BlockSpec((TILE, 8), row),
            pl.BlockSpec((TILE, 2), row),
            whole2((16, 16)), whole2((1, 16)),
            whole2((24, 24)), whole2((1, 24)),
            whole2((16, 32)), whole2((1, 32)),
            whole2((1, 72)), whole2((1, 72)),
            whole3((4, 72, 32)),
        ],
        out_specs=[pl.BlockSpec((TILE, HID), row)] * 4,
        out_shape=[out32] * 4,
    )(x, idr, e0, e1, e2, degp,
      p['W_id'], p['b_id'].reshape(1, -1),
      p['W_emb'], p['b_emb'].reshape(1, -1),
      p['W0'], p['b0'].reshape(1, -1),
      p['ln0_g'].reshape(1, -1), p['ln0_b'].reshape(1, -1),
      p['tag_W'])


def _hop_body(pp_ref, base_ref, degp_ref, u_ref):
    dinv = _dinv_from_degp(degp_ref[...])
    psum = pp_ref[0] + pp_ref[1]
    u_ref[...] = dinv * (base_ref[...] + dinv * psum)


def _hop_combine(pp, base, degp):
    return pl.pallas_call(
        _hop_body,
        grid=(GRID,),
        in_specs=[
            pl.BlockSpec((2, TILE, HID), lambda i: (0, i, 0)),
            pl.BlockSpec((TILE, HID), lambda i: (i, 0)),
            pl.BlockSpec((TILE, 2), lambda i: (i, 0)),
        ],
        out_specs=pl.BlockSpec((TILE, HID), lambda i: (i, 0)),
        out_shape=jax.ShapeDtypeStruct((NP, HID), jnp.float32),
    )(pp, base, degp)


def _epi_body(pp_ref, base_ref, degp_ref, tagb_ref, g_ref, b_ref,
              w1_ref, b1_ref, out_ref):
    dinv = _dinv_from_degp(degp_ref[...])
    y = base_ref[...] + dinv * (pp_ref[0] + pp_ref[1])
    y = jnp.maximum(y + tagb_ref[...], 0.0)
    mu = jnp.mean(y, axis=1, keepdims=True)
    var = jnp.mean(y * y, axis=1, keepdims=True) - mu * mu
    y = (y - mu) * lax.rsqrt(jnp.maximum(var, 0.0) + LN_EPS) * g_ref[...] + b_ref[...]
    o = jnp.dot(y, w1_ref[...], preferred_element_type=jnp.float32) + b1_ref[...]
    m = jnp.max(o, axis=1, keepdims=True)
    z = o - m
    out_ref[...] = z - jnp.log(jnp.sum(jnp.exp(z), axis=1, keepdims=True))


def _epilogue(pp, base0, degp, p):
    whole2 = lambda shape: pl.BlockSpec(shape, lambda i: (0, 0))
    return pl.pallas_call(
        _epi_body,
        grid=(GRID,),
        in_specs=[
            pl.BlockSpec((2, TILE, HID), lambda i: (0, i, 0)),
            pl.BlockSpec((TILE, HID), lambda i: (i, 0)),
            pl.BlockSpec((TILE, 2), lambda i: (i, 0)),
            whole2((1, 32)), whole2((1, 32)), whole2((1, 32)),
            whole2((32, 2)), whole2((1, 2)),
        ],
        out_specs=pl.BlockSpec((TILE, 2), lambda i: (i, 0)),
        out_shape=jax.ShapeDtypeStruct((NP, 2), jnp.float32),
    )(pp, base0, degp,
      p['tag_b'].reshape(1, -1), p['ln1_g'].reshape(1, -1),
      p['ln1_b'].reshape(1, -1), p['W1'], p['b1'].reshape(1, -1))


def _hop_sc_body(u_hbm, srcg_hbm, dstg_hbm, wg_hbm, p_hbm,
                 srci, dsti, wi, rows_a, rows_b, zbuf, acc, sem_a, sem_b):
    core = lax.axis_index("c")
    sub = lax.axis_index("s")
    tile = core * 16 + sub

    zero16 = jnp.zeros((16,), jnp.float32)

    @pl.loop(0, ZROWS)
    def _(r):
        zbuf[r, pl.ds(0, 16)] = zero16
        zbuf[r, pl.ds(16, 16)] = zero16

    @pl.loop(0, RPS // ZROWS)
    def _(t):
        pltpu.sync_copy(zbuf, acc.at[pl.ds(sub * RPS + t * ZROWS, ZROWS)])

    plsc.subcore_barrier()

    def scale_and_scatter(buf, jr):
        idxr = jnp.full((16,), jr, jnp.int32)

        @pl.loop(0, CHUNK)
        def _(c):
            s16 = plsc.load_gather(wi, [idxr, jnp.full((16,), c, jnp.int32)])
            buf[c, pl.ds(0, 16)] = buf[c, pl.ds(0, 16)] * s16
            buf[c, pl.ds(16, 16)] = buf[c, pl.ds(16, 16)] * s16

        pltpu.sync_copy(buf, acc.at[dsti.at[jr]], add=True)

    @pl.loop(0, CPT // BLK)
    def _(bb):
        base = tile * CPT + bb * BLK
        pltpu.sync_copy(srcg_hbm.at[pl.ds(base, BLK)], srci)
        pltpu.sync_copy(dstg_hbm.at[pl.ds(base, BLK)], dsti)
        pltpu.sync_copy(wg_hbm.at[pl.ds(base, BLK)], wi)

        @pl.loop(0, BLK // 2)
        def _(jp):
            jr = jp * 2
            da = pltpu.async_copy(u_hbm.at[srci.at[jr]], rows_a, sem_a)
            db = pltpu.async_copy(u_hbm.at[srci.at[jr + 1]], rows_b, sem_b)
            da.wait()
            scale_and_scatter(rows_a, jr)
            db.wait()
            scale_and_scatter(rows_b, jr + 1)

    plsc.subcore_barrier()
    pltpu.sync_copy(acc.at[pl.ds(sub * RPS, RPS)],
                    p_hbm.at[core].at[pl.ds(sub * RPS, RPS)])


_hop_sc = pl.kernel(
    _hop_sc_body,
    out_type=jax.ShapeDtypeStruct((2, NP, HID), jnp.float32),
    mesh=plsc.VectorSubcoreMesh(core_axis_name="c", subcore_axis_name="s",
                                num_cores=2, num_subcores=16),
    scratch_types=[
        pltpu.VMEM((BLK, CHUNK), jnp.int32),
        pltpu.VMEM((BLK, CHUNK), jnp.int32),
        pltpu.VMEM((BLK, CHUNK), jnp.float32),
        pltpu.VMEM((CHUNK, HID), jnp.float32),
        pltpu.VMEM((CHUNK, HID), jnp.float32),
        pltpu.VMEM((ZROWS, HID), jnp.float32),
        pltpu.VMEM_SHARED((NP, HID), jnp.float32),
        pltpu.SemaphoreType.DMA,
        pltpu.SemaphoreType.DMA,
    ],
    compiler_params=pltpu.CompilerParams(use_tc_tiling_on_sc=False,
                                         needs_layout_passes=False),
)


def kernel(x, edge_index, edge_weight, categories_value, params):
    p = params
    src = edge_index[0]
    dst = edge_index[1]

    idr = jnp.take(p['id_table'], categories_value[:, 0], axis=0)
    e0 = jnp.take(p['emb_tables'][0], categories_value[:, 1], axis=0)
    e1 = jnp.take(p['emb_tables'][1], categories_value[:, 2], axis=0)
    e2 = jnp.take(p['emb_tables'][2], categories_value[:, 3], axis=0)
    deg = jnp.zeros((NN,), jnp.float32).at[dst].add(edge_weight)
    degp = jnp.stack([deg, jnp.zeros_like(deg)], axis=1)

    pad_rows = lambda a: jnp.pad(a, ((0, NP - NN), (0, 0)))
    x_p = pad_rows(x)
    idr_p, e0_p, e1_p, e2_p = map(pad_rows, (idr, e0, e1, e2))
    degp_p = pad_rows(degp)

    base0, base1, base2, u = _front(x_p, idr_p, e0_p, e1_p, e2_p, degp_p, p)

    pad_e = lambda a: jnp.pad(a, (0, EP - EE)).reshape(EP // CHUNK, CHUNK)
    srcg = pad_e(src)
    dstg = pad_e(dst)
    wg = pad_e(edge_weight)

    pp = _hop_sc(u, srcg, dstg, wg)
    u = _hop_combine(pp, base2, degp_p)
    pp = _hop_sc(u, srcg, dstg, wg)
    u = _hop_combine(pp, base1, degp_p)
    pp = _hop_sc(u, srcg, dstg, wg)
    out = _epilogue(pp, base0, degp_p, p)
    return out[:NN]


# speedup vs baseline: 8.2680x; 1.0949x over previous
import functools
import jax
import jax.numpy as jnp
from jax import lax
from jax.experimental import pallas as pl
from jax.experimental.pallas import tpu as pltpu
from jax.experimental.pallas import tpu_sc as plsc

NN = 50000
EE = 800000
NP = 50176
TILE = 512
GRID = NP // TILE
D_ID, D_H, D_E, D_ALL, HID = 16, 32, 24, 72, 32
LN_EPS = 1e-5

NTILES = 32
CHUNK = 128
CPT = 200
EP = NTILES * CPT * CHUNK
RPS = NP // 16
ZROWS = 112
BLK = 40


def _elu(v):
    return jnp.where(v > 0, v, jnp.exp(jnp.minimum(v, 0.0)) - 1.0)


def _dinv_from_degp(degp_blk):
    d = jnp.sum(degp_blk, axis=1, keepdims=True)
    return jnp.where(d > 0, lax.rsqrt(jnp.maximum(d, 1e-30)), 0.0)


def _front_body(x_ref, idr_ref, e0_ref, e1_ref, e2_ref, degp_ref,
                wid_ref, bid_ref, wemb_ref, bemb_ref, w0_ref, b0_ref,
                g_ref, b_ref, tagw_ref,
                base0_ref, base1_ref, base2_ref, u3_ref):
    h = _elu(jnp.dot(x_ref[...], w0_ref[...],
                     preferred_element_type=jnp.float32) + b0_ref[...])
    idp = _elu(jnp.dot(idr_ref[...], wid_ref[...],
                       preferred_element_type=jnp.float32) + bid_ref[...])
    wemb = wemb_ref[...]
    ep = (jnp.dot(e0_ref[...], wemb[0:8, :], preferred_element_type=jnp.float32)
          + jnp.dot(e1_ref[...], wemb[8:16, :], preferred_element_type=jnp.float32)
          + jnp.dot(e2_ref[...], wemb[16:24, :], preferred_element_type=jnp.float32)
          + bemb_ref[...])
    ep = _elu(ep)
    s1 = (jnp.sum(idp, axis=1, keepdims=True)
          + jnp.sum(h, axis=1, keepdims=True)
          + jnp.sum(ep, axis=1, keepdims=True))
    s2 = (jnp.sum(idp * idp, axis=1, keepdims=True)
          + jnp.sum(h * h, axis=1, keepdims=True)
          + jnp.sum(ep * ep, axis=1, keepdims=True))
    mu = s1 / D_ALL
    var = s2 / D_ALL - mu * mu
    inv = lax.rsqrt(jnp.maximum(var, 0.0) + LN_EPS)
    g = g_ref[...]
    bb = b_ref[...]
    idn = (idp - mu) * inv * g[:, 0:16] + bb[:, 0:16]
    hn = (h - mu) * inv * g[:, 16:48] + bb[:, 16:48]
    en = (ep - mu) * inv * g[:, 48:72] + bb[:, 48:72]
    tw = tagw_ref[...]

    def base(k):
        return (jnp.dot(idn, tw[k, 0:16, :], preferred_element_type=jnp.float32)
                + jnp.dot(hn, tw[k, 16:48, :], preferred_element_type=jnp.float32)
                + jnp.dot(en, tw[k, 48:72, :], preferred_element_type=jnp.float32))

    base0_ref[...] = base(0)
    base1_ref[...] = base(1)
    base2_ref[...] = base(2)
    dinv = _dinv_from_degp(degp_ref[...])
    u3_ref[...] = dinv * base(3)


def _front(x, idr, e0, e1, e2, degp, p):
    row = lambda i: (i, 0)
    whole2 = lambda shape: pl.BlockSpec(shape, lambda i: (0, 0))
    whole3 = lambda shape: pl.BlockSpec(shape, lambda i: (0, 0, 0))
    out32 = jax.ShapeDtypeStruct((NP, HID), jnp.float32)
    return pl.pallas_call(
        _front_body,
        grid=(GRID,),
        in_specs=[
            pl.BlockSpec((TILE, 16), row),
            pl.BlockSpec((TILE, 16), row),
            pl.BlockSpec((TILE, 8), row),
            pl.BlockSpec((TILE, 8), row),
            pl.BlockSpec((TILE, 8), row),
            pl.BlockSpec((TILE, 2), row),
            whole2((16, 16)), whole2((1, 16)),
            whole2((24, 24)), whole2((1, 24)),
            whole2((16, 32)), whole2((1, 32)),
            whole2((1, 72)), whole2((1, 72)),
            whole3((4, 72, 32)),
        ],
        out_specs=[pl.BlockSpec((TILE, HID), row)] * 4,
        out_shape=[out32] * 4,
    )(x, idr, e0, e1, e2, degp,
      p['W_id'], p['b_id'].reshape(1, -1),
      p['W_emb'], p['b_emb'].reshape(1, -1),
      p['W0'], p['b0'].reshape(1, -1),
      p['ln0_g'].reshape(1, -1), p['ln0_b'].reshape(1, -1),
      p['tag_W'])


def _hop_body(pp_ref, base_ref, degp_ref, u_ref):
    dinv = _dinv_from_degp(degp_ref[...])
    psum = pp_ref[0] + pp_ref[1]
    u_ref[...] = dinv * (base_ref[...] + dinv * psum)


def _hop_combine(pp, base, degp):
    return pl.pallas_call(
        _hop_body,
        grid=(GRID,),
        in_specs=[
            pl.BlockSpec((2, TILE, HID), lambda i: (0, i, 0)),
            pl.BlockSpec((TILE, HID), lambda i: (i, 0)),
            pl.BlockSpec((TILE, 2), lambda i: (i, 0)),
        ],
        out_specs=pl.BlockSpec((TILE, HID), lambda i: (i, 0)),
        out_shape=jax.ShapeDtypeStruct((NP, HID), jnp.float32),
    )(pp, base, degp)


def _epi_body(pp_ref, base_ref, degp_ref, tagb_ref, g_ref, b_ref,
              w1_ref, b1_ref, out_ref):
    dinv = _dinv_from_degp(degp_ref[...])
    y = base_ref[...] + dinv * (pp_ref[0] + pp_ref[1])
    y = jnp.maximum(y + tagb_ref[...], 0.0)
    mu = jnp.mean(y, axis=1, keepdims=True)
    var = jnp.mean(y * y, axis=1, keepdims=True) - mu * mu
    y = (y - mu) * lax.rsqrt(jnp.maximum(var, 0.0) + LN_EPS) * g_ref[...] + b_ref[...]
    o = jnp.dot(y, w1_ref[...], preferred_element_type=jnp.float32) + b1_ref[...]
    m = jnp.max(o, axis=1, keepdims=True)
    z = o - m
    out_ref[...] = z - jnp.log(jnp.sum(jnp.exp(z), axis=1, keepdims=True))


def _epilogue(pp, base0, degp, p):
    whole2 = lambda shape: pl.BlockSpec(shape, lambda i: (0, 0))
    return pl.pallas_call(
        _epi_body,
        grid=(GRID,),
        in_specs=[
            pl.BlockSpec((2, TILE, HID), lambda i: (0, i, 0)),
            pl.BlockSpec((TILE, HID), lambda i: (i, 0)),
            pl.BlockSpec((TILE, 2), lambda i: (i, 0)),
            whole2((1, 32)), whole2((1, 32)), whole2((1, 32)),
            whole2((32, 2)), whole2((1, 2)),
        ],
        out_specs=pl.BlockSpec((TILE, 2), lambda i: (i, 0)),
        out_shape=jax.ShapeDtypeStruct((NP, 2), jnp.float32),
    )(pp, base0, degp,
      p['tag_b'].reshape(1, -1), p['ln1_g'].reshape(1, -1),
      p['ln1_b'].reshape(1, -1), p['W1'], p['b1'].reshape(1, -1))


def _hop_sc_body(u_hbm, srcg_hbm, dstg_hbm, wg_hbm, p_hbm,
                 srci, dsti, wi, rows_a, rows_b, zbuf, acc, sem_a, sem_b):
    core = lax.axis_index("c")
    sub = lax.axis_index("s")
    tile = core * 16 + sub

    zero16 = jnp.zeros((16,), jnp.float32)

    @pl.loop(0, ZROWS)
    def _(r):
        zbuf[r, pl.ds(0, 16)] = zero16
        zbuf[r, pl.ds(16, 16)] = zero16

    @pl.loop(0, RPS // ZROWS)
    def _(t):
        pltpu.sync_copy(zbuf, acc.at[pl.ds(sub * RPS + t * ZROWS, ZROWS)])

    plsc.subcore_barrier()

    def scale_and_scatter(buf, jr):
        idxr = jnp.full((16,), jr, jnp.int32)

        @plsc.parallel_loop(0, CHUNK, unroll=8)
        def _(c):
            s16 = plsc.load_gather(wi, [idxr, jnp.full((16,), c, jnp.int32)])
            buf[c, pl.ds(0, 16)] = buf[c, pl.ds(0, 16)] * s16
            buf[c, pl.ds(16, 16)] = buf[c, pl.ds(16, 16)] * s16

        pltpu.sync_copy(buf, acc.at[dsti.at[jr]], add=True)

    @pl.loop(0, CPT // BLK)
    def _(bb):
        base = tile * CPT + bb * BLK
        pltpu.sync_copy(srcg_hbm.at[pl.ds(base, BLK)], srci)
        pltpu.sync_copy(dstg_hbm.at[pl.ds(base, BLK)], dsti)
        pltpu.sync_copy(wg_hbm.at[pl.ds(base, BLK)], wi)

        @pl.loop(0, BLK // 2)
        def _(jp):
            jr = jp * 2
            da = pltpu.async_copy(u_hbm.at[srci.at[jr]], rows_a, sem_a)
            db = pltpu.async_copy(u_hbm.at[srci.at[jr + 1]], rows_b, sem_b)
            da.wait()
            scale_and_scatter(rows_a, jr)
            db.wait()
            scale_and_scatter(rows_b, jr + 1)

    plsc.subcore_barrier()
    pltpu.sync_copy(acc.at[pl.ds(sub * RPS, RPS)],
                    p_hbm.at[core].at[pl.ds(sub * RPS, RPS)])


_hop_sc = pl.kernel(
    _hop_sc_body,
    out_type=jax.ShapeDtypeStruct((2, NP, HID), jnp.float32),
    mesh=plsc.VectorSubcoreMesh(core_axis_name="c", subcore_axis_name="s",
                                num_cores=2, num_subcores=16),
    scratch_types=[
        pltpu.VMEM((BLK, CHUNK), jnp.int32),
        pltpu.VMEM((BLK, CHUNK), jnp.int32),
        pltpu.VMEM((BLK, CHUNK), jnp.float32),
        pltpu.VMEM((CHUNK, HID), jnp.float32),
        pltpu.VMEM((CHUNK, HID), jnp.float32),
        pltpu.VMEM((ZROWS, HID), jnp.float32),
        pltpu.VMEM_SHARED((NP, HID), jnp.float32),
        pltpu.SemaphoreType.DMA,
        pltpu.SemaphoreType.DMA,
    ],
    compiler_params=pltpu.CompilerParams(use_tc_tiling_on_sc=False,
                                         needs_layout_passes=False),
)


def kernel(x, edge_index, edge_weight, categories_value, params):
    p = params
    src = edge_index[0]
    dst = edge_index[1]

    idr = jnp.take(p['id_table'], categories_value[:, 0], axis=0)
    e0 = jnp.take(p['emb_tables'][0], categories_value[:, 1], axis=0)
    e1 = jnp.take(p['emb_tables'][1], categories_value[:, 2], axis=0)
    e2 = jnp.take(p['emb_tables'][2], categories_value[:, 3], axis=0)
    deg = jnp.zeros((NN,), jnp.float32).at[dst].add(edge_weight)
    degp = jnp.stack([deg, jnp.zeros_like(deg)], axis=1)

    pad_rows = lambda a: jnp.pad(a, ((0, NP - NN), (0, 0)))
    x_p = pad_rows(x)
    idr_p, e0_p, e1_p, e2_p = map(pad_rows, (idr, e0, e1, e2))
    degp_p = pad_rows(degp)

    base0, base1, base2, u = _front(x_p, idr_p, e0_p, e1_p, e2_p, degp_p, p)

    pad_e = lambda a: jnp.pad(a, (0, EP - EE)).reshape(EP // CHUNK, CHUNK)
    srcg = pad_e(src)
    dstg = pad_e(dst)
    wg = pad_e(edge_weight)

    pp = _hop_sc(u, srcg, dstg, wg)
    u = _hop_combine(pp, base2, degp_p)
    pp = _hop_sc(u, srcg, dstg, wg)
    u = _hop_combine(pp, base1, degp_p)
    pp = _hop_sc(u, srcg, dstg, wg)
    out = _epilogue(pp, base0, degp_p, p)
    return out[:NN]


# speedup vs baseline: 8.6279x; 1.0435x over previous
import functools
import jax
import jax.numpy as jnp
from jax import lax
from jax.experimental import pallas as pl
from jax.experimental.pallas import tpu as pltpu
from jax.experimental.pallas import tpu_sc as plsc

NN = 50000
EE = 800000
NP = 50176
TILE = 512
GRID = NP // TILE
D_ID, D_H, D_E, D_ALL, HID = 16, 32, 24, 72, 32
LN_EPS = 1e-5

NTILES = 32
CHUNK = 128
CPT = 200
EP = NTILES * CPT * CHUNK
RPS = NP // 16
ZROWS = 112
BLK = 16
CA = 256
CB = 2 * CPT - CA


def _elu(v):
    return jnp.where(v > 0, v, jnp.exp(jnp.minimum(v, 0.0)) - 1.0)


def _dinv_from_degp(degp_blk):
    d = jnp.sum(degp_blk, axis=1, keepdims=True)
    return jnp.where(d > 0, lax.rsqrt(jnp.maximum(d, 1e-30)), 0.0)


def _front_body(x_ref, idr_ref, e0_ref, e1_ref, e2_ref, degp_ref,
                wid_ref, bid_ref, wemb_ref, bemb_ref, w0_ref, b0_ref,
                g_ref, b_ref, tagw_ref,
                base0_ref, base1_ref, base2_ref, u3_ref):
    h = _elu(jnp.dot(x_ref[...], w0_ref[...],
                     preferred_element_type=jnp.float32) + b0_ref[...])
    idp = _elu(jnp.dot(idr_ref[...], wid_ref[...],
                       preferred_element_type=jnp.float32) + bid_ref[...])
    wemb = wemb_ref[...]
    ep = (jnp.dot(e0_ref[...], wemb[0:8, :], preferred_element_type=jnp.float32)
          + jnp.dot(e1_ref[...], wemb[8:16, :], preferred_element_type=jnp.float32)
          + jnp.dot(e2_ref[...], wemb[16:24, :], preferred_element_type=jnp.float32)
          + bemb_ref[...])
    ep = _elu(ep)
    s1 = (jnp.sum(idp, axis=1, keepdims=True)
          + jnp.sum(h, axis=1, keepdims=True)
          + jnp.sum(ep, axis=1, keepdims=True))
    s2 = (jnp.sum(idp * idp, axis=1, keepdims=True)
          + jnp.sum(h * h, axis=1, keepdims=True)
          + jnp.sum(ep * ep, axis=1, keepdims=True))
    mu = s1 / D_ALL
    var = s2 / D_ALL - mu * mu
    inv = lax.rsqrt(jnp.maximum(var, 0.0) + LN_EPS)
    g = g_ref[...]
    bb = b_ref[...]
    idn = (idp - mu) * inv * g[:, 0:16] + bb[:, 0:16]
    hn = (h - mu) * inv * g[:, 16:48] + bb[:, 16:48]
    en = (ep - mu) * inv * g[:, 48:72] + bb[:, 48:72]
    tw = tagw_ref[...]

    def base(k):
        return (jnp.dot(idn, tw[k, 0:16, :], preferred_element_type=jnp.float32)
                + jnp.dot(hn, tw[k, 16:48, :], preferred_element_type=jnp.float32)
                + jnp.dot(en, tw[k, 48:72, :], preferred_element_type=jnp.float32))

    base0_ref[...] = base(0)
    base1_ref[...] = base(1)
    base2_ref[...] = base(2)
    dinv = _dinv_from_degp(degp_ref[...])
    u3_ref[...] = dinv * base(3)


def _front(x, idr, e0, e1, e2, degp, p):
    row = lambda i: (i, 0)
    whole2 = lambda shape: pl.BlockSpec(shape, lambda i: (0, 0))
    whole3 = lambda shape: pl.BlockSpec(shape, lambda i: (0, 0, 0))
    out32 = jax.ShapeDtypeStruct((NP, HID), jnp.float32)
    return pl.pallas_call(
        _front_body,
        grid=(GRID,),
        in_specs=[
            pl.BlockSpec((TILE, 16), row),
            pl.BlockSpec((TILE, 16), row),
            pl.BlockSpec((TILE, 8), row),
            pl.BlockSpec((TILE, 8), row),
            pl.BlockSpec((TILE, 8), row),
            pl.BlockSpec((TILE, 2), row),
            whole2((16, 16)), whole2((1, 16)),
            whole2((24, 24)), whole2((1, 24)),
            whole2((16, 32)), whole2((1, 32)),
            whole2((1, 72)), whole2((1, 72)),
            whole3((4, 72, 32)),
        ],
        out_specs=[pl.BlockSpec((TILE, HID), row)] * 4,
        out_shape=[out32] * 4,
    )(x, idr, e0, e1, e2, degp,
      p['W_id'], p['b_id'].reshape(1, -1),
      p['W_emb'], p['b_emb'].reshape(1, -1),
      p['W0'], p['b0'].reshape(1, -1),
      p['ln0_g'].reshape(1, -1), p['ln0_b'].reshape(1, -1),
      p['tag_W'])


def _hop_body(pp_ref, base_ref, degp_ref, u_ref):
    dinv = _dinv_from_degp(degp_ref[...])
    psum = pp_ref[0] + pp_ref[1]
    u_ref[...] = dinv * (base_ref[...] + dinv * psum)


def _hop_combine(pp, base, degp):
    return pl.pallas_call(
        _hop_body,
        grid=(GRID,),
        in_specs=[
            pl.BlockSpec((2, TILE, HID), lambda i: (0, i, 0)),
            pl.BlockSpec((TILE, HID), lambda i: (i, 0)),
            pl.BlockSpec((TILE, 2), lambda i: (i, 0)),
        ],
        out_specs=pl.BlockSpec((TILE, HID), lambda i: (i, 0)),
        out_shape=jax.ShapeDtypeStruct((NP, HID), jnp.float32),
    )(pp, base, degp)


def _epi_body(pp_ref, base_ref, degp_ref, tagb_ref, g_ref, b_ref,
              w1_ref, b1_ref, out_ref):
    dinv = _dinv_from_degp(degp_ref[...])
    y = base_ref[...] + dinv * (pp_ref[0] + pp_ref[1])
    y = jnp.maximum(y + tagb_ref[...], 0.0)
    mu = jnp.mean(y, axis=1, keepdims=True)
    var = jnp.mean(y * y, axis=1, keepdims=True) - mu * mu
    y = (y - mu) * lax.rsqrt(jnp.maximum(var, 0.0) + LN_EPS) * g_ref[...] + b_ref[...]
    o = jnp.dot(y, w1_ref[...], preferred_element_type=jnp.float32) + b1_ref[...]
    m = jnp.max(o, axis=1, keepdims=True)
    z = o - m
    out_ref[...] = z - jnp.log(jnp.sum(jnp.exp(z), axis=1, keepdims=True))


def _epilogue(pp, base0, degp, p):
    whole2 = lambda shape: pl.BlockSpec(shape, lambda i: (0, 0))
    return pl.pallas_call(
        _epi_body,
        grid=(GRID,),
        in_specs=[
            pl.BlockSpec((2, TILE, HID), lambda i: (0, i, 0)),
            pl.BlockSpec((TILE, HID), lambda i: (i, 0)),
            pl.BlockSpec((TILE, 2), lambda i: (i, 0)),
            whole2((1, 32)), whole2((1, 32)), whole2((1, 32)),
            whole2((32, 2)), whole2((1, 2)),
        ],
        out_specs=pl.BlockSpec((TILE, 2), lambda i: (i, 0)),
        out_shape=jax.ShapeDtypeStruct((NP, 2), jnp.float32),
    )(pp, base0, degp,
      p['tag_b'].reshape(1, -1), p['ln1_g'].reshape(1, -1),
      p['ln1_b'].reshape(1, -1), p['W1'], p['b1'].reshape(1, -1))


def _hop_sc_body(u_hbm, srcg_hbm, dstg_hbm, wg_hbm, p_hbm,
                 srci, dsti, wi, rows_a, rows_b, zbuf, acc, sem_a, sem_b):
    core = lax.axis_index("c")
    sub = lax.axis_index("s")
    tile = core * 16 + sub

    zero16 = jnp.zeros((16,), jnp.float32)

    @pl.loop(0, ZROWS)
    def _(r):
        zbuf[r, pl.ds(0, 16)] = zero16
        zbuf[r, pl.ds(16, 16)] = zero16

    @pl.loop(0, RPS // ZROWS)
    def _(t):
        pltpu.sync_copy(zbuf, acc.at[pl.ds(sub * RPS + t * ZROWS, ZROWS)])

    plsc.subcore_barrier()

    def scale_and_scatter(buf, jr):
        idxr = jnp.full((16,), jr, jnp.int32)

        @plsc.parallel_loop(0, CHUNK, unroll=8)
        def _(c):
            s16 = plsc.load_gather(wi, [idxr, jnp.full((16,), c, jnp.int32)])
            buf[c, pl.ds(0, 16)] = buf[c, pl.ds(0, 16)] * s16
            buf[c, pl.ds(16, 16)] = buf[c, pl.ds(16, 16)] * s16

        pltpu.sync_copy(buf, acc.at[dsti.at[jr]], add=True)

    my_cnt = jnp.where(core == 0, CA, CB)
    my_base = jnp.where(core == 0, sub * CA, 16 * CA + sub * CB)

    @pl.loop(0, max(CA, CB) // BLK)
    def _(bb):
        @pl.when(bb * BLK < my_cnt)
        def _():
            base = my_base + bb * BLK
            pltpu.sync_copy(srcg_hbm.at[pl.ds(base, BLK)], srci)
            pltpu.sync_copy(dstg_hbm.at[pl.ds(base, BLK)], dsti)
            pltpu.sync_copy(wg_hbm.at[pl.ds(base, BLK)], wi)

            @pl.loop(0, BLK // 2)
            def _(jp):
                jr = jp * 2
                da = pltpu.async_copy(u_hbm.at[srci.at[jr]], rows_a, sem_a)
                db = pltpu.async_copy(u_hbm.at[srci.at[jr + 1]], rows_b, sem_b)
                da.wait()
                scale_and_scatter(rows_a, jr)
                db.wait()
                scale_and_scatter(rows_b, jr + 1)

    plsc.subcore_barrier()
    pltpu.sync_copy(acc.at[pl.ds(sub * RPS, RPS)],
                    p_hbm.at[core].at[pl.ds(sub * RPS, RPS)])


_hop_sc = pl.kernel(
    _hop_sc_body,
    out_type=jax.ShapeDtypeStruct((2, NP, HID), jnp.float32),
    mesh=plsc.VectorSubcoreMesh(core_axis_name="c", subcore_axis_name="s",
                                num_cores=2, num_subcores=16),
    scratch_types=[
        pltpu.VMEM((BLK, CHUNK), jnp.int32),
        pltpu.VMEM((BLK, CHUNK), jnp.int32),
        pltpu.VMEM((BLK, CHUNK), jnp.float32),
        pltpu.VMEM((CHUNK, HID), jnp.float32),
        pltpu.VMEM((CHUNK, HID), jnp.float32),
        pltpu.VMEM((ZROWS, HID), jnp.float32),
        pltpu.VMEM_SHARED((NP, HID), jnp.float32),
        pltpu.SemaphoreType.DMA,
        pltpu.SemaphoreType.DMA,
    ],
    compiler_params=pltpu.CompilerParams(use_tc_tiling_on_sc=False,
                                         needs_layout_passes=False),
)


def kernel(x, edge_index, edge_weight, categories_value, params):
    p = params
    src = edge_index[0]
    dst = edge_index[1]

    idr = jnp.take(p['id_table'], categories_value[:, 0], axis=0)
    e0 = jnp.take(p['emb_tables'][0], categories_value[:, 1], axis=0)
    e1 = jnp.take(p['emb_tables'][1], categories_value[:, 2], axis=0)
    e2 = jnp.take(p['emb_tables'][2], categories_value[:, 3], axis=0)
    deg = jnp.zeros((NN,), jnp.float32).at[dst].add(edge_weight)
    degp = jnp.stack([deg, jnp.zeros_like(deg)], axis=1)

    pad_rows = lambda a: jnp.pad(a, ((0, NP - NN), (0, 0)))
    x_p = pad_rows(x)
    idr_p, e0_p, e1_p, e2_p = map(pad_rows, (idr, e0, e1, e2))
    degp_p = pad_rows(degp)

    base0, base1, base2, u = _front(x_p, idr_p, e0_p, e1_p, e2_p, degp_p, p)

    pad_e = lambda a: jnp.pad(a, (0, EP - EE)).reshape(EP // CHUNK, CHUNK)
    srcg = pad_e(src)
    dstg = pad_e(dst)
    wg = pad_e(edge_weight)

    pp = _hop_sc(u, srcg, dstg, wg)
    u = _hop_combine(pp, base2, degp_p)
    pp = _hop_sc(u, srcg, dstg, wg)
    u = _hop_combine(pp, base1, degp_p)
    pp = _hop_sc(u, srcg, dstg, wg)
    out = _epilogue(pp, base0, degp_p, p)
    return out[:NN]


# speedup vs baseline: 12.3849x; 1.4354x over previous
import functools
import jax
import jax.numpy as jnp
from jax import lax
from jax.experimental import pallas as pl
from jax.experimental.pallas import tpu as pltpu
from jax.experimental.pallas import tpu_sc as plsc

NN = 50000
EE = 800000
NP = 50176
TILE = 512
GRID = NP // TILE
D_ID, D_H, D_E, D_ALL, HID = 16, 32, 24, 72, 32
LN_EPS = 1e-5

NTILES = 32
CHUNK = 128
CPT = 200
EP = NTILES * CPT * CHUNK
RPS = NP // 16
ZROWS = 112
BLK = 16
CA = 256
CB = 2 * CPT - CA


def _elu(v):
    return jnp.where(v > 0, v, jnp.exp(jnp.minimum(v, 0.0)) - 1.0)


def _dinv_from_degp(degp_blk):
    d = jnp.sum(degp_blk, axis=1, keepdims=True)
    return jnp.where(d > 0, lax.rsqrt(jnp.maximum(d, 1e-30)), 0.0)


def _front_body(x_ref, idr_ref, e0_ref, e1_ref, e2_ref, degp_ref,
                wid_ref, bid_ref, wemb_ref, bemb_ref, w0_ref, b0_ref,
                g_ref, b_ref, tagw_ref,
                base0_ref, base1_ref, base2_ref, u3_ref):
    h = _elu(jnp.dot(x_ref[...], w0_ref[...],
                     preferred_element_type=jnp.float32) + b0_ref[...])
    idp = _elu(jnp.dot(idr_ref[...], wid_ref[...],
                       preferred_element_type=jnp.float32) + bid_ref[...])
    wemb = wemb_ref[...]
    ep = (jnp.dot(e0_ref[:, 0:8], wemb[0:8, :], preferred_element_type=jnp.float32)
          + jnp.dot(e1_ref[:, 0:8], wemb[8:16, :], preferred_element_type=jnp.float32)
          + jnp.dot(e2_ref[:, 0:8], wemb[16:24, :], preferred_element_type=jnp.float32)
          + bemb_ref[...])
    ep = _elu(ep)
    s1 = (jnp.sum(idp, axis=1, keepdims=True)
          + jnp.sum(h, axis=1, keepdims=True)
          + jnp.sum(ep, axis=1, keepdims=True))
    s2 = (jnp.sum(idp * idp, axis=1, keepdims=True)
          + jnp.sum(h * h, axis=1, keepdims=True)
          + jnp.sum(ep * ep, axis=1, keepdims=True))
    mu = s1 / D_ALL
    var = s2 / D_ALL - mu * mu
    inv = lax.rsqrt(jnp.maximum(var, 0.0) + LN_EPS)
    g = g_ref[...]
    bb = b_ref[...]
    idn = (idp - mu) * inv * g[:, 0:16] + bb[:, 0:16]
    hn = (h - mu) * inv * g[:, 16:48] + bb[:, 16:48]
    en = (ep - mu) * inv * g[:, 48:72] + bb[:, 48:72]
    tw = tagw_ref[...]

    def base(k):
        return (jnp.dot(idn, tw[k, 0:16, :], preferred_element_type=jnp.float32)
                + jnp.dot(hn, tw[k, 16:48, :], preferred_element_type=jnp.float32)
                + jnp.dot(en, tw[k, 48:72, :], preferred_element_type=jnp.float32))

    base0_ref[...] = base(0)
    base1_ref[...] = base(1)
    base2_ref[...] = base(2)
    dinv = _dinv_from_degp(degp_ref[...])
    u3_ref[...] = dinv * base(3)


def _front(x, idr, e0, e1, e2, degp, p):
    row = lambda i: (i, 0)
    whole2 = lambda shape: pl.BlockSpec(shape, lambda i: (0, 0))
    whole3 = lambda shape: pl.BlockSpec(shape, lambda i: (0, 0, 0))
    out32 = jax.ShapeDtypeStruct((NP, HID), jnp.float32)
    return pl.pallas_call(
        _front_body,
        grid=(GRID,),
        in_specs=[
            pl.BlockSpec((TILE, 16), row),
            pl.BlockSpec((TILE, 16), row),
            pl.BlockSpec((TILE, 16), row),
            pl.BlockSpec((TILE, 16), row),
            pl.BlockSpec((TILE, 16), row),
            pl.BlockSpec((TILE, 2), row),
            whole2((16, 16)), whole2((1, 16)),
            whole2((24, 24)), whole2((1, 24)),
            whole2((16, 32)), whole2((1, 32)),
            whole2((1, 72)), whole2((1, 72)),
            whole3((4, 72, 32)),
        ],
        out_specs=[pl.BlockSpec((TILE, HID), row)] * 4,
        out_shape=[out32] * 4,
    )(x, idr, e0, e1, e2, degp,
      p['W_id'], p['b_id'].reshape(1, -1),
      p['W_emb'], p['b_emb'].reshape(1, -1),
      p['W0'], p['b0'].reshape(1, -1),
      p['ln0_g'].reshape(1, -1), p['ln0_b'].reshape(1, -1),
      p['tag_W'])


def _hop_body(pp_ref, base_ref, degp_ref, u_ref):
    dinv = _dinv_from_degp(degp_ref[...])
    psum = pp_ref[0] + pp_ref[1]
    u_ref[...] = dinv * (base_ref[...] + dinv * psum)


def _hop_combine(pp, base, degp):
    return pl.pallas_call(
        _hop_body,
        grid=(GRID,),
        in_specs=[
            pl.BlockSpec((2, TILE, HID), lambda i: (0, i, 0)),
            pl.BlockSpec((TILE, HID), lambda i: (i, 0)),
            pl.BlockSpec((TILE, 2), lambda i: (i, 0)),
        ],
        out_specs=pl.BlockSpec((TILE, HID), lambda i: (i, 0)),
        out_shape=jax.ShapeDtypeStruct((NP, HID), jnp.float32),
    )(pp, base, degp)


def _epi_body(pp_ref, base_ref, degp_ref, tagb_ref, g_ref, b_ref,
              w1_ref, b1_ref, out_ref):
    dinv = _dinv_from_degp(degp_ref[...])
    y = base_ref[...] + dinv * (pp_ref[0] + pp_ref[1])
    y = jnp.maximum(y + tagb_ref[...], 0.0)
    mu = jnp.mean(y, axis=1, keepdims=True)
    var = jnp.mean(y * y, axis=1, keepdims=True) - mu * mu
    y = (y - mu) * lax.rsqrt(jnp.maximum(var, 0.0) + LN_EPS) * g_ref[...] + b_ref[...]
    o = jnp.dot(y, w1_ref[...], preferred_element_type=jnp.float32) + b1_ref[...]
    m = jnp.max(o, axis=1, keepdims=True)
    z = o - m
    out_ref[...] = z - jnp.log(jnp.sum(jnp.exp(z), axis=1, keepdims=True))


def _epilogue(pp, base0, degp, p):
    whole2 = lambda shape: pl.BlockSpec(shape, lambda i: (0, 0))
    return pl.pallas_call(
        _epi_body,
        grid=(GRID,),
        in_specs=[
            pl.BlockSpec((2, TILE, HID), lambda i: (0, i, 0)),
            pl.BlockSpec((TILE, HID), lambda i: (i, 0)),
            pl.BlockSpec((TILE, 2), lambda i: (i, 0)),
            whole2((1, 32)), whole2((1, 32)), whole2((1, 32)),
            whole2((32, 2)), whole2((1, 2)),
        ],
        out_specs=pl.BlockSpec((TILE, 2), lambda i: (i, 0)),
        out_shape=jax.ShapeDtypeStruct((NP, 2), jnp.float32),
    )(pp, base0, degp,
      p['tag_b'].reshape(1, -1), p['ln1_g'].reshape(1, -1),
      p['ln1_b'].reshape(1, -1), p['W1'], p['b1'].reshape(1, -1))


def _hop_sc_body(u_hbm, srcg_hbm, dstg_hbm, wg_hbm, p_hbm,
                 srci, dsti, wi, rows_a, rows_b, zbuf, acc, sem_a, sem_b):
    core = lax.axis_index("c")
    sub = lax.axis_index("s")
    tile = core * 16 + sub

    zero16 = jnp.zeros((16,), jnp.float32)

    @pl.loop(0, ZROWS)
    def _(r):
        zbuf[r, pl.ds(0, 16)] = zero16
        zbuf[r, pl.ds(16, 16)] = zero16

    @pl.loop(0, RPS // ZROWS)
    def _(t):
        pltpu.sync_copy(zbuf, acc.at[pl.ds(sub * RPS + t * ZROWS, ZROWS)])

    plsc.subcore_barrier()

    def scale_and_scatter(buf, jr):
        idxr = jnp.full((16,), jr, jnp.int32)

        @plsc.parallel_loop(0, CHUNK, unroll=8)
        def _(c):
            s16 = plsc.load_gather(wi, [idxr, jnp.full((16,), c, jnp.int32)])
            buf[c, pl.ds(0, 16)] = buf[c, pl.ds(0, 16)] * s16
            buf[c, pl.ds(16, 16)] = buf[c, pl.ds(16, 16)] * s16

        pltpu.sync_copy(buf, acc.at[dsti.at[jr]], add=True)

    my_cnt = jnp.where(core == 0, CA, CB)
    my_base = jnp.where(core == 0, sub * CA, 16 * CA + sub * CB)

    @pl.loop(0, max(CA, CB) // BLK)
    def _(bb):
        @pl.when(bb * BLK < my_cnt)
        def _():
            base = my_base + bb * BLK
            pltpu.sync_copy(srcg_hbm.at[pl.ds(base, BLK)], srci)
            pltpu.sync_copy(dstg_hbm.at[pl.ds(base, BLK)], dsti)
            pltpu.sync_copy(wg_hbm.at[pl.ds(base, BLK)], wi)

            @pl.loop(0, BLK // 2)
            def _(jp):
                jr = jp * 2
                da = pltpu.async_copy(u_hbm.at[srci.at[jr]], rows_a, sem_a)
                db = pltpu.async_copy(u_hbm.at[srci.at[jr + 1]], rows_b, sem_b)
                da.wait()
                scale_and_scatter(rows_a, jr)
                db.wait()
                scale_and_scatter(rows_b, jr + 1)

    plsc.subcore_barrier()
    pltpu.sync_copy(acc.at[pl.ds(sub * RPS, RPS)],
                    p_hbm.at[core].at[pl.ds(sub * RPS, RPS)])


_hop_sc = pl.kernel(
    _hop_sc_body,
    out_type=jax.ShapeDtypeStruct((2, NP, HID), jnp.float32),
    mesh=plsc.VectorSubcoreMesh(core_axis_name="c", subcore_axis_name="s",
                                num_cores=2, num_subcores=16),
    scratch_types=[
        pltpu.VMEM((BLK, CHUNK), jnp.int32),
        pltpu.VMEM((BLK, CHUNK), jnp.int32),
        pltpu.VMEM((BLK, CHUNK), jnp.float32),
        pltpu.VMEM((CHUNK, HID), jnp.float32),
        pltpu.VMEM((CHUNK, HID), jnp.float32),
        pltpu.VMEM((ZROWS, HID), jnp.float32),
        pltpu.VMEM_SHARED((NP, HID), jnp.float32),
        pltpu.SemaphoreType.DMA,
        pltpu.SemaphoreType.DMA,
    ],
    compiler_params=pltpu.CompilerParams(use_tc_tiling_on_sc=False,
                                         needs_layout_passes=False),
)


NGRP = 49


def _front_sc_body(idt, et0, et1, et2, cats, dstg_hbm, wg_hbm,
                   idr_o, e0_o, e1_o, e2_o, degp_o,
                   idx8, dsti, wvi, gb_a, gb_b, zb, dacc, sem_a, sem_b):
    core = lax.axis_index("c")
    sub = lax.axis_index("s")
    tile = core * 16 + sub

    zero16 = jnp.zeros((16,), jnp.float32)

    @pl.loop(0, 49)
    def _(r):
        zb[pl.ds(r * 16, 16)] = zero16

    @pl.loop(0, 4)
    def _(t):
        pltpu.sync_copy(zb, dacc.at[pl.ds(sub * RPS + t * 784, 784)])

    plsc.subcore_barrier()

    @pl.loop(0, 25)
    def _(b):
        base = tile * CPT + b * 8
        pltpu.sync_copy(dstg_hbm.at[pl.ds(base, 8)], dsti)
        pltpu.sync_copy(wg_hbm.at[pl.ds(base, 8)], wvi)
        descs = [pltpu.async_copy(wvi.at[r], dacc.at[dsti.at[r]], sem_a,
                                  add=True) for r in range(8)]
        for d in descs:
            d.wait()

    ngrp = jnp.where(tile < 17, 2, 1)
    gbase = jnp.where(tile < 17, 2 * tile, tile + 17)

    @pl.loop(0, 2)
    def _(g):
        @pl.when(g < ngrp)
        def _():
            grp = gbase + g
            for k, (tab, out) in enumerate(
                    ((idt, idr_o), (et0, e0_o), (et1, e1_o), (et2, e2_o))):
                pltpu.sync_copy(cats.at[k].at[pl.ds(grp * 8, 8)], idx8)

                @pl.loop(0, 4)
                def _(rp):
                    r = rp * 2
                    obase = grp * 1024 + r * 128
                    da = pltpu.async_copy(tab.at[idx8.at[r]], gb_a, sem_a)
                    db = pltpu.async_copy(tab.at[idx8.at[r + 1]], gb_b, sem_b)
                    da.wait()
                    pltpu.sync_copy(gb_a, out.at[pl.ds(obase, 128)])
                    db.wait()
                    pltpu.sync_copy(gb_b, out.at[pl.ds(obase + 128, 128)])

    plsc.subcore_barrier()
    pltpu.sync_copy(dacc.at[pl.ds(sub * RPS, RPS)],
                    degp_o.at[core].at[pl.ds(sub * RPS, RPS)])


_front_sc = pl.kernel(
    _front_sc_body,
    out_type=[
        jax.ShapeDtypeStruct((NP, 16), jnp.float32),
        jax.ShapeDtypeStruct((NP, 16), jnp.float32),
        jax.ShapeDtypeStruct((NP, 16), jnp.float32),
        jax.ShapeDtypeStruct((NP, 16), jnp.float32),
        jax.ShapeDtypeStruct((2, NP), jnp.float32),
    ],
    mesh=plsc.VectorSubcoreMesh(core_axis_name="c", subcore_axis_name="s",
                                num_cores=2, num_subcores=16),
    scratch_types=[
        pltpu.VMEM((8, CHUNK), jnp.int32),
        pltpu.VMEM((8, CHUNK), jnp.int32),
        pltpu.VMEM((8, CHUNK), jnp.float32),
        pltpu.VMEM((CHUNK, 16), jnp.float32),
        pltpu.VMEM((CHUNK, 16), jnp.float32),
        pltpu.VMEM((784,), jnp.float32),
        pltpu.VMEM_SHARED((NP,), jnp.float32),
        pltpu.SemaphoreType.DMA,
        pltpu.SemaphoreType.DMA,
    ],
    compiler_params=pltpu.CompilerParams(use_tc_tiling_on_sc=False,
                                         needs_layout_passes=False),
)


def kernel(x, edge_index, edge_weight, categories_value, params):
    p = params
    src = edge_index[0]
    dst = edge_index[1]

    pad_e = lambda a: jnp.pad(a, (0, EP - EE)).reshape(EP // CHUNK, CHUNK)
    srcg = pad_e(src)
    dstg = pad_e(dst)
    wg = pad_e(edge_weight)

    cat_p = jnp.pad(categories_value.T, ((0, 0), (0, NP - NN)))
    cat_p = cat_p.reshape(4, NP // CHUNK, CHUNK)
    et0 = jnp.pad(p['emb_tables'][0], ((0, 0), (0, 8)))
    et1 = jnp.pad(p['emb_tables'][1], ((0, 0), (0, 8)))
    et2 = jnp.pad(p['emb_tables'][2], ((0, 0), (0, 8)))
    idr_p, e0_p, e1_p, e2_p, degp2 = _front_sc(
        p['id_table'], et0, et1, et2, cat_p, dstg, wg)
    degp_p = degp2.T

    x_p = jnp.pad(x, ((0, NP - NN), (0, 0)))
    base0, base1, base2, u = _front(x_p, idr_p, e0_p, e1_p, e2_p, degp_p, p)

    pp = _hop_sc(u, srcg, dstg, wg)
    u = _hop_combine(pp, base2, degp_p)
    pp = _hop_sc(u, srcg, dstg, wg)
    u = _hop_combine(pp, base1, degp_p)
    pp = _hop_sc(u, srcg, dstg, wg)
    out = _epilogue(pp, base0, degp_p, p)
    return out[:NN]


# speedup vs baseline: 13.5545x; 1.0944x over previous
import functools
import jax
import jax.numpy as jnp
from jax import lax
from jax.experimental import pallas as pl
from jax.experimental.pallas import tpu as pltpu
from jax.experimental.pallas import tpu_sc as plsc

NN = 50000
EE = 800000
NP = 50176
TILE = 3584
GRID = NP // TILE
D_ID, D_H, D_E, D_ALL, HID = 16, 32, 24, 72, 32
LN_EPS = 1e-5

NTILES = 32
CHUNK = 128
CPT = 200
EP = NTILES * CPT * CHUNK
RPS = NP // 16
ZROWS = 112
BLK = 16
CA = 256
CB = 2 * CPT - CA


def _elu(v):
    return jnp.where(v > 0, v, jnp.exp(jnp.minimum(v, 0.0)) - 1.0)


def _dinv_from_degp(degp_blk):
    d = jnp.sum(degp_blk, axis=1, keepdims=True)
    return jnp.where(d > 0, lax.rsqrt(jnp.maximum(d, 1e-30)), 0.0)


def _front_body(x_ref, idr_ref, e0_ref, e1_ref, e2_ref, degp_ref,
                wid_ref, bid_ref, wemb_ref, bemb_ref, w0_ref, b0_ref,
                g_ref, b_ref, tagw_ref,
                base0_ref, base1_ref, base2_ref, u3_ref):
    h = _elu(jnp.dot(x_ref[...], w0_ref[...],
                     preferred_element_type=jnp.float32) + b0_ref[...])
    idp = _elu(jnp.dot(idr_ref[...], wid_ref[...],
                       preferred_element_type=jnp.float32) + bid_ref[...])
    wemb = wemb_ref[...]
    ep = (jnp.dot(e0_ref[:, 0:8], wemb[0:8, :], preferred_element_type=jnp.float32)
          + jnp.dot(e1_ref[:, 0:8], wemb[8:16, :], preferred_element_type=jnp.float32)
          + jnp.dot(e2_ref[:, 0:8], wemb[16:24, :], preferred_element_type=jnp.float32)
          + bemb_ref[...])
    ep = _elu(ep)
    s1 = (jnp.sum(idp, axis=1, keepdims=True)
          + jnp.sum(h, axis=1, keepdims=True)
          + jnp.sum(ep, axis=1, keepdims=True))
    s2 = (jnp.sum(idp * idp, axis=1, keepdims=True)
          + jnp.sum(h * h, axis=1, keepdims=True)
          + jnp.sum(ep * ep, axis=1, keepdims=True))
    mu = s1 / D_ALL
    var = s2 / D_ALL - mu * mu
    inv = lax.rsqrt(jnp.maximum(var, 0.0) + LN_EPS)
    g = g_ref[...]
    bb = b_ref[...]
    idn = (idp - mu) * inv * g[:, 0:16] + bb[:, 0:16]
    hn = (h - mu) * inv * g[:, 16:48] + bb[:, 16:48]
    en = (ep - mu) * inv * g[:, 48:72] + bb[:, 48:72]
    tw = tagw_ref[...]

    def base(k):
        return (jnp.dot(idn, tw[k, 0:16, :], preferred_element_type=jnp.float32)
                + jnp.dot(hn, tw[k, 16:48, :], preferred_element_type=jnp.float32)
                + jnp.dot(en, tw[k, 48:72, :], preferred_element_type=jnp.float32))

    base0_ref[...] = base(0)
    base1_ref[...] = base(1)
    base2_ref[...] = base(2)
    dinv = _dinv_from_degp(degp_ref[...])
    u3_ref[...] = dinv * base(3)


def _front(x, idr, e0, e1, e2, degp, p):
    row = lambda i: (i, 0)
    whole2 = lambda shape: pl.BlockSpec(shape, lambda i: (0, 0))
    whole3 = lambda shape: pl.BlockSpec(shape, lambda i: (0, 0, 0))
    out32 = jax.ShapeDtypeStruct((NP, HID), jnp.float32)
    return pl.pallas_call(
        _front_body,
        grid=(GRID,),
        in_specs=[
            pl.BlockSpec((TILE, 16), row),
            pl.BlockSpec((TILE, 16), row),
            pl.BlockSpec((TILE, 16), row),
            pl.BlockSpec((TILE, 16), row),
            pl.BlockSpec((TILE, 16), row),
            pl.BlockSpec((TILE, 2), row),
            whole2((16, 16)), whole2((1, 16)),
            whole2((24, 24)), whole2((1, 24)),
            whole2((16, 32)), whole2((1, 32)),
            whole2((1, 72)), whole2((1, 72)),
            whole3((4, 72, 32)),
        ],
        out_specs=[pl.BlockSpec((TILE, HID), row)] * 4,
        out_shape=[out32] * 4,
    )(x, idr, e0, e1, e2, degp,
      p['W_id'], p['b_id'].reshape(1, -1),
      p['W_emb'], p['b_emb'].reshape(1, -1),
      p['W0'], p['b0'].reshape(1, -1),
      p['ln0_g'].reshape(1, -1), p['ln0_b'].reshape(1, -1),
      p['tag_W'])


def _hop_body(pp_ref, base_ref, degp_ref, u_ref):
    dinv = _dinv_from_degp(degp_ref[...])
    psum = pp_ref[0] + pp_ref[1]
    u_ref[...] = dinv * (base_ref[...] + dinv * psum)


def _hop_combine(pp, base, degp):
    return pl.pallas_call(
        _hop_body,
        grid=(GRID,),
        in_specs=[
            pl.BlockSpec((2, TILE, HID), lambda i: (0, i, 0)),
            pl.BlockSpec((TILE, HID), lambda i: (i, 0)),
            pl.BlockSpec((TILE, 2), lambda i: (i, 0)),
        ],
        out_specs=pl.BlockSpec((TILE, HID), lambda i: (i, 0)),
        out_shape=jax.ShapeDtypeStruct((NP, HID), jnp.float32),
    )(pp, base, degp)


def _epi_body(pp_ref, base_ref, degp_ref, tagb_ref, g_ref, b_ref,
              w1_ref, b1_ref, out_ref):
    dinv = _dinv_from_degp(degp_ref[...])
    y = base_ref[...] + dinv * (pp_ref[0] + pp_ref[1])
    y = jnp.maximum(y + tagb_ref[...], 0.0)
    mu = jnp.mean(y, axis=1, keepdims=True)
    var = jnp.mean(y * y, axis=1, keepdims=True) - mu * mu
    y = (y - mu) * lax.rsqrt(jnp.maximum(var, 0.0) + LN_EPS) * g_ref[...] + b_ref[...]
    o = jnp.dot(y, w1_ref[...], preferred_element_type=jnp.float32) + b1_ref[...]
    m = jnp.max(o, axis=1, keepdims=True)
    z = o - m
    out_ref[...] = z - jnp.log(jnp.sum(jnp.exp(z), axis=1, keepdims=True))


def _epilogue(pp, base0, degp, p):
    whole2 = lambda shape: pl.BlockSpec(shape, lambda i: (0, 0))
    return pl.pallas_call(
        _epi_body,
        grid=(GRID,),
        in_specs=[
            pl.BlockSpec((2, TILE, HID), lambda i: (0, i, 0)),
            pl.BlockSpec((TILE, HID), lambda i: (i, 0)),
            pl.BlockSpec((TILE, 2), lambda i: (i, 0)),
            whole2((1, 32)), whole2((1, 32)), whole2((1, 32)),
            whole2((32, 2)), whole2((1, 2)),
        ],
        out_specs=pl.BlockSpec((TILE, 2), lambda i: (i, 0)),
        out_shape=jax.ShapeDtypeStruct((NP, 2), jnp.float32),
    )(pp, base0, degp,
      p['tag_b'].reshape(1, -1), p['ln1_g'].reshape(1, -1),
      p['ln1_b'].reshape(1, -1), p['W1'], p['b1'].reshape(1, -1))


def _hop_sc_body(u_hbm, srcg_hbm, dstg_hbm, wg_hbm, p_hbm,
                 srci, dsti, wi, rows_a, rows_b, zbuf, acc, sem_a, sem_b):
    core = lax.axis_index("c")
    sub = lax.axis_index("s")
    tile = core * 16 + sub

    zero16 = jnp.zeros((16,), jnp.float32)

    @pl.loop(0, ZROWS)
    def _(r):
        zbuf[r, pl.ds(0, 16)] = zero16
        zbuf[r, pl.ds(16, 16)] = zero16

    @pl.loop(0, RPS // ZROWS)
    def _(t):
        pltpu.sync_copy(zbuf, acc.at[pl.ds(sub * RPS + t * ZROWS, ZROWS)])

    plsc.subcore_barrier()

    def scale_and_scatter(buf, jr):
        idxr = jnp.full((16,), jr, jnp.int32)

        @plsc.parallel_loop(0, CHUNK, unroll=8)
        def _(c):
            s16 = plsc.load_gather(wi, [idxr, jnp.full((16,), c, jnp.int32)])
            buf[c, pl.ds(0, 16)] = buf[c, pl.ds(0, 16)] * s16
            buf[c, pl.ds(16, 16)] = buf[c, pl.ds(16, 16)] * s16

        pltpu.sync_copy(buf, acc.at[dsti.at[jr]], add=True)

    my_cnt = jnp.where(core == 0, CA, CB)
    my_base = jnp.where(core == 0, sub * CA, 16 * CA + sub * CB)

    @pl.loop(0, max(CA, CB) // BLK)
    def _(bb):
        @pl.when(bb * BLK < my_cnt)
        def _():
            base = my_base + bb * BLK
            pltpu.sync_copy(srcg_hbm.at[pl.ds(base, BLK)], srci)
            pltpu.sync_copy(dstg_hbm.at[pl.ds(base, BLK)], dsti)
            pltpu.sync_copy(wg_hbm.at[pl.ds(base, BLK)], wi)

            @pl.loop(0, BLK // 2)
            def _(jp):
                jr = jp * 2
                da = pltpu.async_copy(u_hbm.at[srci.at[jr]], rows_a, sem_a)
                db = pltpu.async_copy(u_hbm.at[srci.at[jr + 1]], rows_b, sem_b)
                da.wait()
                scale_and_scatter(rows_a, jr)
                db.wait()
                scale_and_scatter(rows_b, jr + 1)

    plsc.subcore_barrier()
    pltpu.sync_copy(acc.at[pl.ds(sub * RPS, RPS)],
                    p_hbm.at[core].at[pl.ds(sub * RPS, RPS)])


_hop_sc = pl.kernel(
    _hop_sc_body,
    out_type=jax.ShapeDtypeStruct((2, NP, HID), jnp.float32),
    mesh=plsc.VectorSubcoreMesh(core_axis_name="c", subcore_axis_name="s",
                                num_cores=2, num_subcores=16),
    scratch_types=[
        pltpu.VMEM((BLK, CHUNK), jnp.int32),
        pltpu.VMEM((BLK, CHUNK), jnp.int32),
        pltpu.VMEM((BLK, CHUNK), jnp.float32),
        pltpu.VMEM((CHUNK, HID), jnp.float32),
        pltpu.VMEM((CHUNK, HID), jnp.float32),
        pltpu.VMEM((ZROWS, HID), jnp.float32),
        pltpu.VMEM_SHARED((NP, HID), jnp.float32),
        pltpu.SemaphoreType.DMA,
        pltpu.SemaphoreType.DMA,
    ],
    compiler_params=pltpu.CompilerParams(use_tc_tiling_on_sc=False,
                                         needs_layout_passes=False),
)


NGRP = 49


def _front_sc_body(idt, et0, et1, et2, cats, dstg_hbm, wg_hbm,
                   idr_o, e0_o, e1_o, e2_o, degp_o,
                   idx8, dsti, wvi, gb_a, gb_b, zb, dacc, sem_a, sem_b):
    core = lax.axis_index("c")
    sub = lax.axis_index("s")
    tile = core * 16 + sub

    zero16 = jnp.zeros((16,), jnp.float32)

    @pl.loop(0, 49)
    def _(r):
        zb[pl.ds(r * 16, 16)] = zero16

    @pl.loop(0, 4)
    def _(t):
        pltpu.sync_copy(zb, dacc.at[pl.ds(sub * RPS + t * 784, 784)])

    plsc.subcore_barrier()

    @pl.loop(0, 25)
    def _(b):
        base = tile * CPT + b * 8
        pltpu.sync_copy(dstg_hbm.at[pl.ds(base, 8)], dsti)
        pltpu.sync_copy(wg_hbm.at[pl.ds(base, 8)], wvi)
        descs = [pltpu.async_copy(wvi.at[r], dacc.at[dsti.at[r]], sem_a,
                                  add=True) for r in range(8)]
        for d in descs:
            d.wait()

    ngrp = jnp.where(tile < 17, 2, 1)
    gbase = jnp.where(tile < 17, 2 * tile, tile + 17)

    @pl.loop(0, 2)
    def _(g):
        @pl.when(g < ngrp)
        def _():
            grp = gbase + g
            for k, (tab, out) in enumerate(
                    ((idt, idr_o), (et0, e0_o), (et1, e1_o), (et2, e2_o))):
                pltpu.sync_copy(cats.at[k].at[pl.ds(grp * 8, 8)], idx8)

                @pl.loop(0, 4)
                def _(rp):
                    r = rp * 2
                    obase = grp * 1024 + r * 128
                    da = pltpu.async_copy(tab.at[idx8.at[r]], gb_a, sem_a)
                    db = pltpu.async_copy(tab.at[idx8.at[r + 1]], gb_b, sem_b)
                    da.wait()
                    pltpu.sync_copy(gb_a, out.at[pl.ds(obase, 128)])
                    db.wait()
                    pltpu.sync_copy(gb_b, out.at[pl.ds(obase + 128, 128)])

    plsc.subcore_barrier()
    pltpu.sync_copy(dacc.at[pl.ds(sub * RPS, RPS)],
                    degp_o.at[core].at[pl.ds(sub * RPS, RPS)])


_front_sc = pl.kernel(
    _front_sc_body,
    out_type=[
        jax.ShapeDtypeStruct((NP, 16), jnp.float32),
        jax.ShapeDtypeStruct((NP, 16), jnp.float32),
        jax.ShapeDtypeStruct((NP, 16), jnp.float32),
        jax.ShapeDtypeStruct((NP, 16), jnp.float32),
        jax.ShapeDtypeStruct((2, NP), jnp.float32),
    ],
    mesh=plsc.VectorSubcoreMesh(core_axis_name="c", subcore_axis_name="s",
                                num_cores=2, num_subcores=16),
    scratch_types=[
        pltpu.VMEM((8, CHUNK), jnp.int32),
        pltpu.VMEM((8, CHUNK), jnp.int32),
        pltpu.VMEM((8, CHUNK), jnp.float32),
        pltpu.VMEM((CHUNK, 16), jnp.float32),
        pltpu.VMEM((CHUNK, 16), jnp.float32),
        pltpu.VMEM((784,), jnp.float32),
        pltpu.VMEM_SHARED((NP,), jnp.float32),
        pltpu.SemaphoreType.DMA,
        pltpu.SemaphoreType.DMA,
    ],
    compiler_params=pltpu.CompilerParams(use_tc_tiling_on_sc=False,
                                         needs_layout_passes=False),
)


def kernel(x, edge_index, edge_weight, categories_value, params):
    p = params
    src = edge_index[0]
    dst = edge_index[1]

    pad_e = lambda a: jnp.pad(a, (0, EP - EE)).reshape(EP // CHUNK, CHUNK)
    srcg = pad_e(src)
    dstg = pad_e(dst)
    wg = pad_e(edge_weight)

    cat_p = jnp.pad(categories_value.T, ((0, 0), (0, NP - NN)))
    cat_p = cat_p.reshape(4, NP // CHUNK, CHUNK)
    et0 = jnp.pad(p['emb_tables'][0], ((0, 0), (0, 8)))
    et1 = jnp.pad(p['emb_tables'][1], ((0, 0), (0, 8)))
    et2 = jnp.pad(p['emb_tables'][2], ((0, 0), (0, 8)))
    idr_p, e0_p, e1_p, e2_p, degp2 = _front_sc(
        p['id_table'], et0, et1, et2, cat_p, dstg, wg)
    degp_p = degp2.T

    x_p = jnp.pad(x, ((0, NP - NN), (0, 0)))
    base0, base1, base2, u = _front(x_p, idr_p, e0_p, e1_p, e2_p, degp_p, p)

    pp = _hop_sc(u, srcg, dstg, wg)
    u = _hop_combine(pp, base2, degp_p)
    pp = _hop_sc(u, srcg, dstg, wg)
    u = _hop_combine(pp, base1, degp_p)
    pp = _hop_sc(u, srcg, dstg, wg)
    out = _epilogue(pp, base0, degp_p, p)
    return out[:NN]
